# baseline (speedup 1.0000x reference)
import functools

import jax
import jax.numpy as jnp
from jax import lax
from jax.experimental import pallas as pl
from jax.experimental.pallas import tpu as pltpu
from jax.experimental.pallas import tpu_sc as plsc

D_MODEL = 64
SCALE = 8.0

NC = 2
NS = 16
LANES = 16
NW = NC * NS

SUP = 1024
GCH = 128
NG = SUP // GCH


@functools.lru_cache(maxsize=None)
def _make_lookup(n):
    b_per_w = n // NW
    n_sup = b_per_w // SUP
    mesh = plsc.VectorSubcoreMesh(
        core_axis_name="c", subcore_axis_name="s",
        num_cores=NC, num_subcores=NS)

    @functools.partial(
        pl.kernel,
        mesh=mesh,
        out_type=jax.ShapeDtypeStruct((n, D_MODEL), jnp.float32),
        scratch_types=[
            pltpu.VMEM((SUP,), jnp.int32),
            pltpu.VMEM((SUP, D_MODEL), jnp.float32),
            pltpu.SemaphoreType.DMA,
        ],
        compiler_params=pltpu.CompilerParams(use_tc_tiling_on_sc=False),
    )
    def lookup(table_hbm, idx_hbm, out_hbm, idx_v, rows_v, sem):
        wid = lax.axis_index("s") * NC + lax.axis_index("c")
        base = wid * b_per_w

        def sup_body(g, carry):
            off = base + g * SUP
            pltpu.sync_copy(idx_hbm.at[pl.ds(off, SUP)], idx_v)
            copies = [
                pltpu.async_copy(
                    table_hbm.at[idx_v.at[pl.ds(j * GCH, GCH)]],
                    rows_v.at[pl.ds(j * GCH, GCH)],
                    sem,
                )
                for j in range(NG)
            ]
            for cp in copies:
                cp.wait()

            def scale_row(r, c2):
                for c in range(D_MODEL // LANES):
                    rows_v[r, pl.ds(c * LANES, LANES)] = (
                        rows_v[r, pl.ds(c * LANES, LANES)] * SCALE)
                return c2

            lax.fori_loop(0, SUP, scale_row, 0)
            pltpu.sync_copy(rows_v, out_hbm.at[pl.ds(off, SUP)])
            return carry

        lax.fori_loop(0, n_sup, sup_body, 0)

    return lookup


def kernel(x, table):
    b, l = x.shape
    idx = x.reshape(b * l).astype(jnp.int32)
    out = _make_lookup(b * l)(table, idx)
    return out.reshape(b, l, D_MODEL)
